# Initial kernel scaffold; baseline (speedup 1.0000x reference)
import functools
import jax
import jax.numpy as jnp
from jax import lax
from jax.experimental import pallas as pl
from jax.experimental.pallas import tpu as pltpu
from jax.experimental.pallas import tpu_sc as plsc

N = 10000
E = 320000
D = 128
NW = 32
C = 125
TCH = E // C
CPT = TCH // NW
RPS = N // 16

_mesh = functools.partial(
    plsc.VectorSubcoreMesh, core_axis_name="c", subcore_axis_name="s")


def _wid():
    return lax.axis_index("s") * 2 + lax.axis_index("c")


@functools.partial(
    pl.kernel,
    out_type=jax.ShapeDtypeStruct((2, N, 16), jnp.float32),
    mesh=_mesh(),
    scratch_types=[
        pltpu.VMEM((CPT, C), jnp.int32),
        pltpu.VMEM((C, 16), jnp.float32),
        pltpu.VMEM((RPS, 16), jnp.float32),
        pltpu.VMEM_SHARED((N, 16), jnp.float32),
    ],
)
def _deg_sc(dstm, out, didx, ones_v, stage, acc):
    cid = lax.axis_index("c")
    sid = lax.axis_index("s")
    wid = _wid()
    one = jnp.full((16,), 1.0, jnp.float32)
    zero = jnp.zeros((16,), jnp.float32)

    def f1(r, _):
        ones_v[r] = one
        return 0
    lax.fori_loop(0, C, f1, 0)

    def f0(r, _):
        stage[r] = zero
        return 0
    lax.fori_loop(0, RPS, f0, 0)

    pltpu.sync_copy(stage, acc.at[pl.ds(sid * RPS, RPS)])
    pltpu.sync_copy(dstm.at[pl.ds(wid * CPT, CPT)], didx)
    plsc.subcore_barrier()

    def chunk(c, _):
        pltpu.sync_copy(ones_v, acc.at[didx.at[c]], add=True)
        return 0
    lax.fori_loop(0, CPT, chunk, 0)

    plsc.subcore_barrier()
    pltpu.sync_copy(acc.at[pl.ds(sid * RPS, RPS)], stage)
    pltpu.sync_copy(stage, out.at[cid, pl.ds(sid * RPS, RPS)])


@functools.partial(
    pl.kernel,
    out_type=jax.ShapeDtypeStruct((2, N, D), jnp.float32),
    mesh=_mesh(),
    scratch_types=[
        pltpu.VMEM((CPT, C), jnp.int32),
        pltpu.VMEM((CPT, C), jnp.int32),
        pltpu.VMEM((C, D), jnp.float32),
        pltpu.VMEM((C, D), jnp.float32),
        pltpu.VMEM_SHARED((N, D), jnp.float32),
        pltpu.SemaphoreType.DMA,
    ],
)
def _seg_sc(table, srcm, dstm, out, sidx, didx, rows, zbuf, acc, sem):
    cid = lax.axis_index("c")
    sid = lax.axis_index("s")
    wid = _wid()
    zero = jnp.zeros((16,), jnp.float32)

    def zr(r, _):
        for j in range(D // 16):
            zbuf[r, pl.ds(j * 16, 16)] = zero
        return 0
    lax.fori_loop(0, C, zr, 0)

    def zc(k, _):
        pltpu.sync_copy(zbuf, acc.at[pl.ds(sid * RPS + k * C, C)])
        return 0
    lax.fori_loop(0, RPS // C, zc, 0)

    pltpu.sync_copy(srcm.at[pl.ds(wid * CPT, CPT)], sidx)
    pltpu.sync_copy(dstm.at[pl.ds(wid * CPT, CPT)], didx)
    plsc.subcore_barrier()

    def chunk(c, _):
        pltpu.async_copy(table.at[sidx.at[c]], rows, sem).wait()
        pltpu.sync_copy(rows, acc.at[didx.at[c]], add=True)
        return 0
    lax.fori_loop(0, CPT, chunk, 0)

    plsc.subcore_barrier()

    def dump(k, _):
        r0 = sid * RPS + k * C
        pltpu.sync_copy(acc.at[pl.ds(r0, C)], zbuf)
        pltpu.sync_copy(zbuf, out.at[cid, pl.ds(r0, C)])
        return 0
    lax.fori_loop(0, RPS // C, dump, 0)


@functools.partial(
    pl.kernel,
    out_type=jax.ShapeDtypeStruct((E, D), jnp.float32),
    mesh=_mesh(),
    scratch_types=[
        pltpu.VMEM((CPT, C), jnp.int32),
        pltpu.VMEM((CPT, C), jnp.int32),
        pltpu.VMEM((C, D), jnp.float32),
        pltpu.VMEM((C, D), jnp.float32),
        pltpu.SemaphoreType.DMA,
        pltpu.SemaphoreType.DMA,
    ],
)
def _edge_sc(xr, xcb, rowm, colm, out, ridx, cidx, buf_a, buf_b, sem1, sem2):
    wid = _wid()
    pltpu.sync_copy(rowm.at[pl.ds(wid * CPT, CPT)], ridx)
    pltpu.sync_copy(colm.at[pl.ds(wid * CPT, CPT)], cidx)

    def chunk(c, _):
        cp1 = pltpu.async_copy(xr.at[ridx.at[c]], buf_a, sem1)
        cp2 = pltpu.async_copy(xcb.at[cidx.at[c]], buf_b, sem2)
        cp1.wait()
        cp2.wait()

        def rowf(e, _):
            for j in range(D // 16):
                s = pl.ds(j * 16, 16)
                buf_a[e, s] = jnp.maximum(buf_a[e, s] + buf_b[e, s], 0.0)
            return 0
        lax.fori_loop(0, C, rowf, 0)

        pltpu.sync_copy(buf_a, out.at[pl.ds((wid * CPT + c) * C, C)])
        return 0
    lax.fori_loop(0, CPT, chunk, 0)


_R = 2000


def _full(shape):
    nd = len(shape)
    return pl.BlockSpec(shape, lambda i, _n=nd: (0,) * _n)


def _t1_body(degp, inp, wa1, ba1, wa2, ba2, w2, hs_o, dis_o):
    d = degp[:]
    deg = d[0, :, 0:1] + d[1, :, 0:1] + 1.0
    dis = lax.rsqrt(deg)
    ai = inp[:, 0:8]
    mf = inp[:, 8:]
    af = jnp.maximum(jnp.dot(ai, wa1[:], preferred_element_type=jnp.float32)
                     + ba1[:][None, :], 0.0)
    af = jnp.dot(af, wa2[:], preferred_element_type=jnp.float32) + ba2[:][None, :]
    x0 = jnp.concatenate([mf, af], axis=1)
    hs_o[:] = jnp.dot(x0, w2[:], preferred_element_type=jnp.float32) * dis
    dis_o[:] = dis


def _t1(deg_parts, inputs, Wa1, ba1, Wa2, ba2, W2):
    f = inputs.shape[1]
    return pl.pallas_call(
        lambda degp, inp, a1, b1, a2, b2, w2, hs_o, dis_o: _t1_body(
            degp, inp[:], a1, b1, a2, b2, w2, hs_o, dis_o),
        grid=N // _R,
        in_specs=[
            pl.BlockSpec((2, _R, 16), lambda i: (0, i, 0)),
            pl.BlockSpec((_R, f), lambda i: (i, 0)),
            _full(Wa1.shape), _full(ba1.shape), _full(Wa2.shape),
            _full(ba2.shape), _full(W2.shape),
        ],
        out_specs=[
            pl.BlockSpec((_R, D), lambda i: (i, 0)),
            pl.BlockSpec((_R, 1), lambda i: (i, 0)),
        ],
        out_shape=[
            jax.ShapeDtypeStruct((N, D), jnp.float32),
            jax.ShapeDtypeStruct((N, 1), jnp.float32),
        ],
    )(deg_parts, inputs, Wa1, ba1, Wa2, ba2, W2)


def _t2_body(segp, hs1, dis, b2, w3, hs2_o):
    s = segp[:]
    seg = s[0] + s[1]
    x1 = jnp.maximum(dis[:] * (seg + hs1[:]) + b2[:][None, :], 0.0)
    hs2_o[:] = jnp.dot(x1, w3[:], preferred_element_type=jnp.float32) * dis[:]


def _t2(segp1, hs1, dis, b2, W3):
    return pl.pallas_call(
        _t2_body,
        grid=N // _R,
        in_specs=[
            pl.BlockSpec((2, _R, D), lambda i: (0, i, 0)),
            pl.BlockSpec((_R, D), lambda i: (i, 0)),
            pl.BlockSpec((_R, 1), lambda i: (i, 0)),
            _full(b2.shape), _full(W3.shape),
        ],
        out_specs=[pl.BlockSpec((_R, D), lambda i: (i, 0))],
        out_shape=[jax.ShapeDtypeStruct((N, D), jnp.float32)],
    )(segp1, hs1, dis, b2, W3)[0]


def _t3_body(segp, hs2, dis, inp, b3, wc1, bc1, wc2, bc2, wb1, bb1, a_m, b_m,
             we1, be1, x_o, pr_o, bx_o, xr_o, xc_o):
    s = segp[:]
    seg = s[0] + s[1]
    x = dis[:] * (seg + hs2[:]) + b3[:][None, :]
    x_o[:] = x
    p1 = jnp.maximum(jnp.dot(x, wc1[:], preferred_element_type=jnp.float32)
                     + bc1[:][None, :], 0.0)
    pr_o[:] = jnp.dot(p1, wc2[:], preferred_element_type=jnp.float32) + bc2[:][None, :]
    q1 = jnp.maximum(jnp.dot(x, wb1[:], preferred_element_type=jnp.float32)
                     + bb1[:][None, :], 0.0)
    q2 = jnp.dot(jnp.dot(q1, a_m[:], preferred_element_type=jnp.float32),
                 b_m[:], preferred_element_type=jnp.float32)
    bx_o[:] = jnp.tanh(q2) + inp[:, 1:5]
    w_full = we1[:]
    xr_o[:] = jnp.dot(x, w_full[0:D, :], preferred_element_type=jnp.float32)
    xc_o[:] = (jnp.dot(x, w_full[D:, :], preferred_element_type=jnp.float32)
               + be1[:][None, :])


def _t3(segp2, hs2, dis, inputs, b3, Wc1, bc1, Wc2, bc2, Wb1, bb1, A, B,
        We1, be1):
    f = inputs.shape[1]
    return pl.pallas_call(
        _t3_body,
        grid=N // _R,
        in_specs=[
            pl.BlockSpec((2, _R, D), lambda i: (0, i, 0)),
            pl.BlockSpec((_R, D), lambda i: (i, 0)),
            pl.BlockSpec((_R, 1), lambda i: (i, 0)),
            pl.BlockSpec((_R, f), lambda i: (i, 0)),
            _full(b3.shape), _full(Wc1.shape), _full(bc1.shape),
            _full(Wc2.shape), _full(bc2.shape), _full(Wb1.shape),
            _full(bb1.shape), _full(A.shape), _full(B.shape),
            _full(We1.shape), _full(be1.shape),
        ],
        out_specs=[
            pl.BlockSpec((_R, D), lambda i: (i, 0)),
            pl.BlockSpec((_R, 10), lambda i: (i, 0)),
            pl.BlockSpec((_R, 4), lambda i: (i, 0)),
            pl.BlockSpec((_R, D), lambda i: (i, 0)),
            pl.BlockSpec((_R, D), lambda i: (i, 0)),
        ],
        out_shape=[
            jax.ShapeDtypeStruct((N, D), jnp.float32),
            jax.ShapeDtypeStruct((N, 10), jnp.float32),
            jax.ShapeDtypeStruct((N, 4), jnp.float32),
            jax.ShapeDtypeStruct((N, D), jnp.float32),
            jax.ShapeDtypeStruct((N, D), jnp.float32),
        ],
    )(segp2, hs2, dis, inputs, b3, Wc1, bc1, Wc2, bc2, Wb1, bb1, A, B,
      We1, be1)


_R4 = 4000


def _t4_body(h1, we2, be2, we3, be3, out):
    h2 = jnp.maximum(jnp.dot(h1[:], we2[:], preferred_element_type=jnp.float32)
                     + be2[:][None, :], 0.0)
    z = jnp.dot(h2, we3[:], preferred_element_type=jnp.float32) + be3[:][None, :]
    out[:] = 1.0 / (1.0 + jnp.exp(-z))


def _t4(h1, We2, be2, We3, be3):
    return pl.pallas_call(
        _t4_body,
        grid=E // _R4,
        in_specs=[
            pl.BlockSpec((_R4, D), lambda i: (i, 0)),
            _full(We2.shape), _full(be2.shape), _full(We3.shape),
            _full(be3.shape),
        ],
        out_specs=[pl.BlockSpec((_R4, 1), lambda i: (i, 0))],
        out_shape=[jax.ShapeDtypeStruct((E, 1), jnp.float32)],
    )(h1, We2, be2, We3, be3)[0]


def kernel(inputs, edge_index, Wa1, ba1, Wa2, ba2, W2, b2, W3, b3, Wc1, bc1,
           Wc2, bc2, Wb1, bb1, A, B, We1, be1, We2, be2, We3, be3):
    src = edge_index[0]
    dst = edge_index[1]
    srcm = src.reshape(TCH, C)
    dstm = dst.reshape(TCH, C)

    deg_parts = _deg_sc(dstm)
    hs1, dis = _t1(deg_parts, inputs, Wa1, ba1, Wa2, ba2, W2)
    segp1 = _seg_sc(hs1, srcm, dstm)
    hs2 = _t2(segp1, hs1, dis, b2, W3)
    segp2 = _seg_sc(hs2, srcm, dstm)
    x, predict, box, xr, xc = _t3(segp2, hs2, dis, inputs, b3, Wc1, bc1,
                                  Wc2, bc2, Wb1, bb1, A, B, We1, be1)
    h1 = _edge_sc(xr, xc, srcm, dstm)
    edge = _t4(h1, We2, be2, We3, be3)
    return predict, box, edge, x


# speedup vs baseline: 4.1404x; 4.1404x over previous
import functools
import jax
import jax.numpy as jnp
from jax import lax
from jax.experimental import pallas as pl
from jax.experimental.pallas import tpu as pltpu
from jax.experimental.pallas import tpu_sc as plsc

N = 10000
E = 320000
D = 128
NW = 32
C = 128
TCH = 2560
CPT = TCH // NW
EP = TCH * C
NP = N + 16

_mesh = functools.partial(
    plsc.VectorSubcoreMesh, core_axis_name="c", subcore_axis_name="s")


def _wid():
    return lax.axis_index("s") * 2 + lax.axis_index("c")


@functools.partial(
    pl.kernel,
    out_type=jax.ShapeDtypeStruct((2, NP, D), jnp.float32),
    mesh=_mesh(),
    scratch_types=[
        pltpu.VMEM((C,), jnp.int32),
        pltpu.VMEM((C, D), jnp.float32),
        pltpu.VMEM_SHARED((NP, D), jnp.float32),
    ],
)
def _deg_sc(dstm, zin, ones_in, out, didx, ones_v, acc):
    cid = lax.axis_index("c")
    sid = lax.axis_index("s")
    wid = _wid()

    @pl.when(sid == 0)
    def _zero():
        pltpu.sync_copy(zin, acc)

    pltpu.sync_copy(ones_in, ones_v)
    plsc.subcore_barrier()

    def chunk(c, _):
        pltpu.sync_copy(dstm.at[wid * CPT + c], didx)
        pltpu.sync_copy(ones_v, acc.at[didx], add=True)
        return 0
    lax.fori_loop(0, CPT, chunk, 0)

    plsc.subcore_barrier()

    @pl.when(sid == 0)
    def _dump():
        pltpu.sync_copy(acc, out.at[cid])


@functools.partial(
    pl.kernel,
    out_type=jax.ShapeDtypeStruct((2, NP, D), jnp.float32),
    mesh=_mesh(),
    scratch_types=[
        pltpu.VMEM((C,), jnp.int32),
        pltpu.VMEM((C,), jnp.int32),
        pltpu.VMEM((C, D), jnp.float32),
        pltpu.VMEM_SHARED((NP, D), jnp.float32),
        pltpu.SemaphoreType.DMA,
    ],
)
def _seg_sc(table, srcm, dstm, zin, out, sidx, didx, rows, acc, sem):
    cid = lax.axis_index("c")
    sid = lax.axis_index("s")
    wid = _wid()

    @pl.when(sid == 0)
    def _zero():
        pltpu.sync_copy(zin, acc)

    plsc.subcore_barrier()

    def chunk(c, _):
        pltpu.sync_copy(srcm.at[wid * CPT + c], sidx)
        pltpu.sync_copy(dstm.at[wid * CPT + c], didx)
        pltpu.async_copy(table.at[sidx], rows, sem).wait()
        pltpu.sync_copy(rows, acc.at[didx], add=True)
        return 0
    lax.fori_loop(0, CPT, chunk, 0)

    plsc.subcore_barrier()

    @pl.when(sid == 0)
    def _dump():
        pltpu.sync_copy(acc, out.at[cid])


@functools.partial(
    pl.kernel,
    out_type=[
        jax.ShapeDtypeStruct((EP, D), jnp.float32),
        jax.ShapeDtypeStruct((EP, D), jnp.float32),
    ],
    mesh=_mesh(),
    scratch_types=[
        pltpu.VMEM((C,), jnp.int32),
        pltpu.VMEM((C,), jnp.int32),
        pltpu.VMEM((C, D), jnp.float32),
        pltpu.VMEM((C, D), jnp.float32),
        pltpu.SemaphoreType.DMA,
        pltpu.SemaphoreType.DMA,
    ],
)
def _edge_sc(xr, xcb, rowm, colm, ga, gb, ridx, cidx, buf_a, buf_b,
             sem1, sem2):
    wid = _wid()

    def chunk(c, _):
        gc = wid * CPT + c
        pltpu.sync_copy(rowm.at[gc], ridx)
        pltpu.sync_copy(colm.at[gc], cidx)
        cp1 = pltpu.async_copy(xr.at[ridx], buf_a, sem1)
        cp2 = pltpu.async_copy(xcb.at[cidx], buf_b, sem2)
        cp1.wait()
        cp2.wait()
        pltpu.sync_copy(buf_a, ga.at[pl.ds(gc * C, C)])
        pltpu.sync_copy(buf_b, gb.at[pl.ds(gc * C, C)])
        return 0
    lax.fori_loop(0, CPT, chunk, 0)


_R = 2000


def _full(shape):
    nd = len(shape)
    return pl.BlockSpec(shape, lambda i, _n=nd: (0,) * _n)


def _t1_body(degp, inp, wa1, ba1, wa2, ba2, w2, hs_o, dis_o):
    deg = degp[0, :, 0:1] + degp[1, :, 0:1] + 1.0
    dis = lax.rsqrt(deg)
    ai = inp[:, 0:8]
    mf = inp[:, 8:]
    af = jnp.maximum(jnp.dot(ai, wa1[:], preferred_element_type=jnp.float32)
                     + ba1[:][None, :], 0.0)
    af = jnp.dot(af, wa2[:], preferred_element_type=jnp.float32) + ba2[:][None, :]
    x0 = jnp.concatenate([mf, af], axis=1)
    hs_o[:] = jnp.dot(x0, w2[:], preferred_element_type=jnp.float32) * dis
    dis_o[:] = dis


def _t1(deg_parts, inputs, Wa1, ba1, Wa2, ba2, W2):
    f = inputs.shape[1]
    return pl.pallas_call(
        _t1_body,
        grid=N // _R,
        in_specs=[
            pl.BlockSpec((2, _R, D), lambda i: (0, i, 0)),
            pl.BlockSpec((_R, f), lambda i: (i, 0)),
            _full(Wa1.shape), _full(ba1.shape), _full(Wa2.shape),
            _full(ba2.shape), _full(W2.shape),
        ],
        out_specs=[
            pl.BlockSpec((_R, D), lambda i: (i, 0)),
            pl.BlockSpec((_R, 1), lambda i: (i, 0)),
        ],
        out_shape=[
            jax.ShapeDtypeStruct((N, D), jnp.float32),
            jax.ShapeDtypeStruct((N, 1), jnp.float32),
        ],
    )(deg_parts, inputs, Wa1, ba1, Wa2, ba2, W2)


def _t2_body(segp, hs1, dis, b2, w3, hs2_o):
    seg = segp[0] + segp[1]
    x1 = jnp.maximum(dis[:] * (seg + hs1[:]) + b2[:][None, :], 0.0)
    hs2_o[:] = jnp.dot(x1, w3[:], preferred_element_type=jnp.float32) * dis[:]


def _t2(segp1, hs1, dis, b2, W3):
    return pl.pallas_call(
        _t2_body,
        grid=N // _R,
        in_specs=[
            pl.BlockSpec((2, _R, D), lambda i: (0, i, 0)),
            pl.BlockSpec((_R, D), lambda i: (i, 0)),
            pl.BlockSpec((_R, 1), lambda i: (i, 0)),
            _full(b2.shape), _full(W3.shape),
        ],
        out_specs=[pl.BlockSpec((_R, D), lambda i: (i, 0))],
        out_shape=[jax.ShapeDtypeStruct((N, D), jnp.float32)],
    )(segp1, hs1, dis, b2, W3)[0]


def _t3_body(segp, hs2, dis, inp, b3, wc1, bc1, wc2, bc2, wb1, bb1, a_m, b_m,
             we1, be1, x_o, pr_o, bx_o, xr_o, xc_o):
    seg = segp[0] + segp[1]
    x = dis[:] * (seg + hs2[:]) + b3[:][None, :]
    x_o[:] = x
    p1 = jnp.maximum(jnp.dot(x, wc1[:], preferred_element_type=jnp.float32)
                     + bc1[:][None, :], 0.0)
    pr_o[:] = jnp.dot(p1, wc2[:], preferred_element_type=jnp.float32) + bc2[:][None, :]
    q1 = jnp.maximum(jnp.dot(x, wb1[:], preferred_element_type=jnp.float32)
                     + bb1[:][None, :], 0.0)
    q2 = jnp.dot(jnp.dot(q1, a_m[:], preferred_element_type=jnp.float32),
                 b_m[:], preferred_element_type=jnp.float32)
    bx_o[:] = jnp.tanh(q2) + inp[:, 1:5]
    w_full = we1[:]
    xr_o[:] = jnp.dot(x, w_full[0:D, :], preferred_element_type=jnp.float32)
    xc_o[:] = (jnp.dot(x, w_full[D:, :], preferred_element_type=jnp.float32)
               + be1[:][None, :])


def _t3(segp2, hs2, dis, inputs, b3, Wc1, bc1, Wc2, bc2, Wb1, bb1, A, B,
        We1, be1):
    f = inputs.shape[1]
    return pl.pallas_call(
        _t3_body,
        grid=N // _R,
        in_specs=[
            pl.BlockSpec((2, _R, D), lambda i: (0, i, 0)),
            pl.BlockSpec((_R, D), lambda i: (i, 0)),
            pl.BlockSpec((_R, 1), lambda i: (i, 0)),
            pl.BlockSpec((_R, f), lambda i: (i, 0)),
            _full(b3.shape), _full(Wc1.shape), _full(bc1.shape),
            _full(Wc2.shape), _full(bc2.shape), _full(Wb1.shape),
            _full(bb1.shape), _full(A.shape), _full(B.shape),
            _full(We1.shape), _full(be1.shape),
        ],
        out_specs=[
            pl.BlockSpec((_R, D), lambda i: (i, 0)),
            pl.BlockSpec((_R, 10), lambda i: (i, 0)),
            pl.BlockSpec((_R, 4), lambda i: (i, 0)),
            pl.BlockSpec((_R, D), lambda i: (i, 0)),
            pl.BlockSpec((_R, D), lambda i: (i, 0)),
        ],
        out_shape=[
            jax.ShapeDtypeStruct((N, D), jnp.float32),
            jax.ShapeDtypeStruct((N, 10), jnp.float32),
            jax.ShapeDtypeStruct((N, 4), jnp.float32),
            jax.ShapeDtypeStruct((N, D), jnp.float32),
            jax.ShapeDtypeStruct((N, D), jnp.float32),
        ],
    )(segp2, hs2, dis, inputs, b3, Wc1, bc1, Wc2, bc2, Wb1, bb1, A, B,
      We1, be1)


_R4 = 4000


def _t4_body(ga, gb, we2, be2, we3, be3, out):
    h1 = jnp.maximum(ga[:] + gb[:], 0.0)
    h2 = jnp.maximum(jnp.dot(h1, we2[:], preferred_element_type=jnp.float32)
                     + be2[:][None, :], 0.0)
    z = jnp.dot(h2, we3[:], preferred_element_type=jnp.float32) + be3[:][None, :]
    out[:] = 1.0 / (1.0 + jnp.exp(-z))


def _t4(ga, gb, We2, be2, We3, be3):
    return pl.pallas_call(
        _t4_body,
        grid=E // _R4,
        in_specs=[
            pl.BlockSpec((_R4, D), lambda i: (i, 0)),
            pl.BlockSpec((_R4, D), lambda i: (i, 0)),
            _full(We2.shape), _full(be2.shape), _full(We3.shape),
            _full(be3.shape),
        ],
        out_specs=[pl.BlockSpec((_R4, 1), lambda i: (i, 0))],
        out_shape=[jax.ShapeDtypeStruct((E, 1), jnp.float32)],
    )(ga, gb, We2, be2, We3, be3)[0]


def kernel(inputs, edge_index, Wa1, ba1, Wa2, ba2, W2, b2, W3, b3, Wc1, bc1,
           Wc2, bc2, Wb1, bb1, A, B, We1, be1, We2, be2, We3, be3):
    src = edge_index[0]
    dst = edge_index[1]
    pad0 = jnp.zeros((EP - E,), jnp.int32)
    srcm = jnp.concatenate([src, pad0]).reshape(TCH, C)
    dst_scat = jnp.concatenate(
        [dst, jnp.full((EP - E,), N, jnp.int32)]).reshape(TCH, C)
    dst_gath = jnp.concatenate([dst, pad0]).reshape(TCH, C)
    zin = jnp.zeros((NP, D), jnp.float32)
    ones_in = jnp.ones((C, D), jnp.float32)

    deg_parts = _deg_sc(dst_scat, zin, ones_in)
    hs1, dis = _t1(deg_parts, inputs, Wa1, ba1, Wa2, ba2, W2)
    segp1 = _seg_sc(hs1, srcm, dst_scat, zin)
    hs2 = _t2(segp1, hs1, dis, b2, W3)
    segp2 = _seg_sc(hs2, srcm, dst_scat, zin)
    x, predict, box, xr, xc = _t3(segp2, hs2, dis, inputs, b3, Wc1, bc1,
                                  Wc2, bc2, Wb1, bb1, A, B, We1, be1)
    ga, gb = _edge_sc(xr, xc, srcm, dst_gath)
    edge = _t4(ga, gb, We2, be2, We3, be3)
    return predict, box, edge, x


# speedup vs baseline: 4.4020x; 1.0632x over previous
import functools
import jax
import jax.numpy as jnp
from jax import lax
from jax.experimental import pallas as pl
from jax.experimental.pallas import tpu as pltpu
from jax.experimental.pallas import tpu_sc as plsc

N = 10000
E = 320000
D = 128
NW = 32
C = 128
TCH = 2560
CPT = TCH // NW
EP = TCH * C
NP = N + 16

_mesh = functools.partial(
    plsc.VectorSubcoreMesh, core_axis_name="c", subcore_axis_name="s")


def _wid():
    return lax.axis_index("s") * 2 + lax.axis_index("c")


@functools.partial(
    pl.kernel,
    out_type=jax.ShapeDtypeStruct((2, NP, D), jnp.float32),
    mesh=_mesh(),
    scratch_types=[
        pltpu.VMEM((2 * C,), jnp.int32),
        pltpu.VMEM((2 * C, D), jnp.float32),
        pltpu.VMEM_SHARED((NP, D), jnp.float32),
    ],
)
def _deg_sc(dstm, zin, ones_in, out, didx, ones_v, acc):
    cid = lax.axis_index("c")
    sid = lax.axis_index("s")
    wid = _wid()

    @pl.when(sid == 0)
    def _zero():
        pltpu.sync_copy(zin, acc)

    pltpu.sync_copy(ones_in, ones_v)
    plsc.subcore_barrier()

    def chunk(c, _):
        e0 = wid * (CPT * C) + c * (2 * C)
        pltpu.sync_copy(dstm.at[pl.ds(e0, 2 * C)], didx)
        pltpu.sync_copy(ones_v, acc.at[didx], add=True)
        return 0
    lax.fori_loop(0, CPT // 2, chunk, 0)

    plsc.subcore_barrier()

    @pl.when(sid == 0)
    def _dump():
        pltpu.sync_copy(acc, out.at[cid])


@functools.partial(
    pl.kernel,
    out_type=jax.ShapeDtypeStruct((2, NP, D), jnp.float32),
    mesh=_mesh(),
    scratch_types=[
        pltpu.VMEM((2 * C,), jnp.int32),
        pltpu.VMEM((2 * C,), jnp.int32),
        pltpu.VMEM((2 * C, D), jnp.float32),
        pltpu.VMEM_SHARED((NP, D), jnp.float32),
        pltpu.SemaphoreType.DMA,
    ],
)
def _seg_sc(table, srcm, dstm, zin, out, sidx, didx, rows, acc, sem):
    cid = lax.axis_index("c")
    sid = lax.axis_index("s")
    wid = _wid()

    @pl.when(sid == 0)
    def _zero():
        pltpu.sync_copy(zin, acc)

    plsc.subcore_barrier()

    def chunk(c, _):
        e0 = wid * (CPT * C) + c * (2 * C)
        pltpu.sync_copy(srcm.at[pl.ds(e0, 2 * C)], sidx)
        pltpu.sync_copy(dstm.at[pl.ds(e0, 2 * C)], didx)
        pltpu.async_copy(table.at[sidx], rows, sem).wait()
        pltpu.sync_copy(rows, acc.at[didx], add=True)
        return 0
    lax.fori_loop(0, CPT // 2, chunk, 0)

    plsc.subcore_barrier()

    @pl.when(sid == 0)
    def _dump():
        pltpu.sync_copy(acc, out.at[cid])


@functools.partial(
    pl.kernel,
    out_type=[
        jax.ShapeDtypeStruct((EP, D), jnp.float32),
        jax.ShapeDtypeStruct((EP, D), jnp.float32),
    ],
    mesh=_mesh(),
    scratch_types=[
        pltpu.VMEM((2 * C,), jnp.int32),
        pltpu.VMEM((2 * C,), jnp.int32),
        pltpu.VMEM((2 * C, D), jnp.float32),
        pltpu.VMEM((2 * C, D), jnp.float32),
        pltpu.SemaphoreType.DMA,
        pltpu.SemaphoreType.DMA,
    ],
)
def _edge_sc(xr, xcb, rowm, colm, ga, gb, ridx, cidx, buf_a, buf_b,
             sem1, sem2):
    wid = _wid()

    def chunk(c, _):
        e0 = wid * (CPT * C) + c * (2 * C)
        pltpu.sync_copy(rowm.at[pl.ds(e0, 2 * C)], ridx)
        pltpu.sync_copy(colm.at[pl.ds(e0, 2 * C)], cidx)
        cp1 = pltpu.async_copy(xr.at[ridx], buf_a, sem1)
        cp2 = pltpu.async_copy(xcb.at[cidx], buf_b, sem2)
        cp1.wait()
        cp2.wait()
        pltpu.sync_copy(buf_a, ga.at[pl.ds(e0, 2 * C)])
        pltpu.sync_copy(buf_b, gb.at[pl.ds(e0, 2 * C)])
        return 0
    lax.fori_loop(0, CPT // 2, chunk, 0)


_R = 2000


def _full(shape):
    nd = len(shape)
    return pl.BlockSpec(shape, lambda i, _n=nd: (0,) * _n)


def _t1_body(degp, inp, wa1, ba1, wa2, ba2, w2, hs_o, dis_o):
    deg = degp[0, :, 0:1] + degp[1, :, 0:1] + 1.0
    dis = lax.rsqrt(deg)
    ai = inp[:, 0:8]
    mf = inp[:, 8:]
    af = jnp.maximum(jnp.dot(ai, wa1[:], preferred_element_type=jnp.float32)
                     + ba1[:][None, :], 0.0)
    af = jnp.dot(af, wa2[:], preferred_element_type=jnp.float32) + ba2[:][None, :]
    x0 = jnp.concatenate([mf, af], axis=1)
    hs_o[:] = jnp.dot(x0, w2[:], preferred_element_type=jnp.float32) * dis
    dis_o[:] = dis


def _t1(deg_parts, inputs, Wa1, ba1, Wa2, ba2, W2):
    f = inputs.shape[1]
    return pl.pallas_call(
        _t1_body,
        grid=N // _R,
        in_specs=[
            pl.BlockSpec((2, _R, D), lambda i: (0, i, 0)),
            pl.BlockSpec((_R, f), lambda i: (i, 0)),
            _full(Wa1.shape), _full(ba1.shape), _full(Wa2.shape),
            _full(ba2.shape), _full(W2.shape),
        ],
        out_specs=[
            pl.BlockSpec((_R, D), lambda i: (i, 0)),
            pl.BlockSpec((_R, 1), lambda i: (i, 0)),
        ],
        out_shape=[
            jax.ShapeDtypeStruct((N, D), jnp.float32),
            jax.ShapeDtypeStruct((N, 1), jnp.float32),
        ],
    )(deg_parts, inputs, Wa1, ba1, Wa2, ba2, W2)


def _t2_body(segp, hs1, dis, b2, w3, hs2_o):
    seg = segp[0] + segp[1]
    x1 = jnp.maximum(dis[:] * (seg + hs1[:]) + b2[:][None, :], 0.0)
    hs2_o[:] = jnp.dot(x1, w3[:], preferred_element_type=jnp.float32) * dis[:]


def _t2(segp1, hs1, dis, b2, W3):
    return pl.pallas_call(
        _t2_body,
        grid=N // _R,
        in_specs=[
            pl.BlockSpec((2, _R, D), lambda i: (0, i, 0)),
            pl.BlockSpec((_R, D), lambda i: (i, 0)),
            pl.BlockSpec((_R, 1), lambda i: (i, 0)),
            _full(b2.shape), _full(W3.shape),
        ],
        out_specs=[pl.BlockSpec((_R, D), lambda i: (i, 0))],
        out_shape=[jax.ShapeDtypeStruct((N, D), jnp.float32)],
    )(segp1, hs1, dis, b2, W3)[0]


def _t3_body(segp, hs2, dis, inp, b3, wc1, bc1, wc2, bc2, wb1, bb1, a_m, b_m,
             we1, be1, x_o, pr_o, bx_o, xr_o, xc_o):
    seg = segp[0] + segp[1]
    x = dis[:] * (seg + hs2[:]) + b3[:][None, :]
    x_o[:] = x
    p1 = jnp.maximum(jnp.dot(x, wc1[:], preferred_element_type=jnp.float32)
                     + bc1[:][None, :], 0.0)
    pr_o[:] = jnp.dot(p1, wc2[:], preferred_element_type=jnp.float32) + bc2[:][None, :]
    q1 = jnp.maximum(jnp.dot(x, wb1[:], preferred_element_type=jnp.float32)
                     + bb1[:][None, :], 0.0)
    q2 = jnp.dot(jnp.dot(q1, a_m[:], preferred_element_type=jnp.float32),
                 b_m[:], preferred_element_type=jnp.float32)
    bx_o[:] = jnp.tanh(q2) + inp[:, 1:5]
    w_full = we1[:]
    xr_o[:] = jnp.dot(x, w_full[0:D, :], preferred_element_type=jnp.float32)
    xc_o[:] = (jnp.dot(x, w_full[D:, :], preferred_element_type=jnp.float32)
               + be1[:][None, :])


def _t3(segp2, hs2, dis, inputs, b3, Wc1, bc1, Wc2, bc2, Wb1, bb1, A, B,
        We1, be1):
    f = inputs.shape[1]
    return pl.pallas_call(
        _t3_body,
        grid=N // _R,
        in_specs=[
            pl.BlockSpec((2, _R, D), lambda i: (0, i, 0)),
            pl.BlockSpec((_R, D), lambda i: (i, 0)),
            pl.BlockSpec((_R, 1), lambda i: (i, 0)),
            pl.BlockSpec((_R, f), lambda i: (i, 0)),
            _full(b3.shape), _full(Wc1.shape), _full(bc1.shape),
            _full(Wc2.shape), _full(bc2.shape), _full(Wb1.shape),
            _full(bb1.shape), _full(A.shape), _full(B.shape),
            _full(We1.shape), _full(be1.shape),
        ],
        out_specs=[
            pl.BlockSpec((_R, D), lambda i: (i, 0)),
            pl.BlockSpec((_R, 10), lambda i: (i, 0)),
            pl.BlockSpec((_R, 4), lambda i: (i, 0)),
            pl.BlockSpec((_R, D), lambda i: (i, 0)),
            pl.BlockSpec((_R, D), lambda i: (i, 0)),
        ],
        out_shape=[
            jax.ShapeDtypeStruct((N, D), jnp.float32),
            jax.ShapeDtypeStruct((N, 10), jnp.float32),
            jax.ShapeDtypeStruct((N, 4), jnp.float32),
            jax.ShapeDtypeStruct((N, D), jnp.float32),
            jax.ShapeDtypeStruct((N, D), jnp.float32),
        ],
    )(segp2, hs2, dis, inputs, b3, Wc1, bc1, Wc2, bc2, Wb1, bb1, A, B,
      We1, be1)


_R4 = 4000


def _t4_body(ga, gb, we2, be2, we3, be3, out):
    h1 = jnp.maximum(ga[:] + gb[:], 0.0)
    h2 = jnp.maximum(jnp.dot(h1, we2[:], preferred_element_type=jnp.float32)
                     + be2[:][None, :], 0.0)
    z = jnp.dot(h2, we3[:], preferred_element_type=jnp.float32) + be3[:][None, :]
    out[:] = 1.0 / (1.0 + jnp.exp(-z))


def _t4(ga, gb, We2, be2, We3, be3):
    return pl.pallas_call(
        _t4_body,
        grid=E // _R4,
        in_specs=[
            pl.BlockSpec((_R4, D), lambda i: (i, 0)),
            pl.BlockSpec((_R4, D), lambda i: (i, 0)),
            _full(We2.shape), _full(be2.shape), _full(We3.shape),
            _full(be3.shape),
        ],
        out_specs=[pl.BlockSpec((_R4, 1), lambda i: (i, 0))],
        out_shape=[jax.ShapeDtypeStruct((E, 1), jnp.float32)],
    )(ga, gb, We2, be2, We3, be3)[0]


def kernel(inputs, edge_index, Wa1, ba1, Wa2, ba2, W2, b2, W3, b3, Wc1, bc1,
           Wc2, bc2, Wb1, bb1, A, B, We1, be1, We2, be2, We3, be3):
    src = edge_index[0]
    dst = edge_index[1]
    pad0 = jnp.zeros((EP - E,), jnp.int32)
    srcm = jnp.concatenate([src, pad0])
    dst_scat = jnp.concatenate([dst, jnp.full((EP - E,), N, jnp.int32)])
    dst_gath = jnp.concatenate([dst, pad0])
    zin = jnp.zeros((NP, D), jnp.float32)
    ones_in = jnp.ones((2 * C, D), jnp.float32)

    deg_parts = _deg_sc(dst_scat, zin, ones_in)
    hs1, dis = _t1(deg_parts, inputs, Wa1, ba1, Wa2, ba2, W2)
    segp1 = _seg_sc(hs1, srcm, dst_scat, zin)
    hs2 = _t2(segp1, hs1, dis, b2, W3)
    segp2 = _seg_sc(hs2, srcm, dst_scat, zin)
    x, predict, box, xr, xc = _t3(segp2, hs2, dis, inputs, b3, Wc1, bc1,
                                  Wc2, bc2, Wb1, bb1, A, B, We1, be1)
    ga, gb = _edge_sc(xr, xc, srcm, dst_gath)
    edge = _t4(ga, gb, We2, be2, We3, be3)
    return predict, box, edge, x


# speedup vs baseline: 4.8839x; 1.1095x over previous
import functools
import jax
import jax.numpy as jnp
from jax import lax
from jax.experimental import pallas as pl
from jax.experimental.pallas import tpu as pltpu
from jax.experimental.pallas import tpu_sc as plsc

N = 10000
E = 320000
D = 128
NW = 32
C = 128
TCH = 2560
CPT = TCH // NW
EP = TCH * C
NP = N + 16

_mesh = functools.partial(
    plsc.VectorSubcoreMesh, core_axis_name="c", subcore_axis_name="s")


def _wid():
    return lax.axis_index("s") * 2 + lax.axis_index("c")


@functools.partial(
    pl.kernel,
    out_type=jax.ShapeDtypeStruct((2, NP, D), jnp.float32),
    mesh=_mesh(),
    scratch_types=[
        pltpu.VMEM((2 * C,), jnp.int32),
        pltpu.VMEM((2 * C, D), jnp.float32),
        pltpu.VMEM_SHARED((NP, D), jnp.float32),
    ],
)
def _deg_sc(dstm, zin, ones_in, out, didx, ones_v, acc):
    cid = lax.axis_index("c")
    sid = lax.axis_index("s")
    wid = _wid()

    @pl.when(sid == 0)
    def _zero():
        pltpu.sync_copy(zin, acc)

    pltpu.sync_copy(ones_in, ones_v)
    plsc.subcore_barrier()

    def chunk(c, _):
        e0 = wid * (CPT * C) + c * (2 * C)
        pltpu.sync_copy(dstm.at[pl.ds(e0, 2 * C)], didx)
        pltpu.sync_copy(ones_v, acc.at[didx], add=True)
        return 0
    lax.fori_loop(0, CPT // 2, chunk, 0)

    plsc.subcore_barrier()

    @pl.when(sid == 0)
    def _dump():
        pltpu.sync_copy(acc, out.at[cid])


@functools.partial(
    pl.kernel,
    out_type=jax.ShapeDtypeStruct((2, NP, D), jnp.float32),
    mesh=_mesh(),
    scratch_types=[
        pltpu.VMEM((C,), jnp.int32),
        pltpu.VMEM((C,), jnp.int32),
        pltpu.VMEM((C,), jnp.int32),
        pltpu.VMEM((C,), jnp.int32),
        pltpu.VMEM((C, D), jnp.float32),
        pltpu.VMEM((C, D), jnp.float32),
        pltpu.VMEM_SHARED((NP, D), jnp.float32),
        pltpu.SemaphoreType.DMA,
        pltpu.SemaphoreType.DMA,
    ],
)
def _seg_sc(table, srcm, dstm, zin, out, sidx_a, didx_a, sidx_b, didx_b,
            rows_a, rows_b, acc, sem_a, sem_b):
    cid = lax.axis_index("c")
    sid = lax.axis_index("s")
    wid = _wid()
    base = wid * (CPT * C)
    last = base + (CPT - 1) * C

    @pl.when(sid == 0)
    def _zero():
        pltpu.sync_copy(zin, acc)

    plsc.subcore_barrier()

    pltpu.sync_copy(srcm.at[pl.ds(base, C)], sidx_a)
    pltpu.sync_copy(dstm.at[pl.ds(base, C)], didx_a)
    pltpu.async_copy(table.at[sidx_a], rows_a, sem_a)

    def pair(k, _):
        e_b = base + (2 * k + 1) * C
        e_a2 = jnp.minimum(base + (2 * k + 2) * C, last)
        pltpu.sync_copy(srcm.at[pl.ds(e_b, C)], sidx_b)
        pltpu.sync_copy(dstm.at[pl.ds(e_b, C)], didx_b)
        pltpu.make_async_copy(table.at[sidx_a], rows_a, sem_a).wait()
        pltpu.async_copy(table.at[sidx_b], rows_b, sem_b)
        pltpu.sync_copy(rows_a, acc.at[didx_a], add=True)
        pltpu.sync_copy(srcm.at[pl.ds(e_a2, C)], sidx_a)
        pltpu.sync_copy(dstm.at[pl.ds(e_a2, C)], didx_a)
        pltpu.async_copy(table.at[sidx_a], rows_a, sem_a)
        pltpu.make_async_copy(table.at[sidx_b], rows_b, sem_b).wait()
        pltpu.sync_copy(rows_b, acc.at[didx_b], add=True)
        return 0
    lax.fori_loop(0, CPT // 2, pair, 0)

    pltpu.make_async_copy(table.at[sidx_a], rows_a, sem_a).wait()
    plsc.subcore_barrier()

    @pl.when(sid == 0)
    def _dump():
        pltpu.sync_copy(acc, out.at[cid])


@functools.partial(
    pl.kernel,
    out_type=[
        jax.ShapeDtypeStruct((EP, D), jnp.float32),
        jax.ShapeDtypeStruct((EP, D), jnp.float32),
    ],
    mesh=_mesh(),
    scratch_types=[
        pltpu.VMEM((C,), jnp.int32),
        pltpu.VMEM((C,), jnp.int32),
        pltpu.VMEM((C,), jnp.int32),
        pltpu.VMEM((C,), jnp.int32),
        pltpu.VMEM((C, D), jnp.float32),
        pltpu.VMEM((C, D), jnp.float32),
        pltpu.VMEM((C, D), jnp.float32),
        pltpu.VMEM((C, D), jnp.float32),
        pltpu.SemaphoreType.DMA,
        pltpu.SemaphoreType.DMA,
        pltpu.SemaphoreType.DMA,
        pltpu.SemaphoreType.DMA,
        pltpu.SemaphoreType.DMA,
        pltpu.SemaphoreType.DMA,
    ],
)
def _edge_sc(xr, xcb, rowm, colm, ga, gb, ridx_a, cidx_a, ridx_b, cidx_b,
             buf_ra, buf_ca, buf_rb, buf_cb, sg1, sg2, sg3, sg4, sw1, sw2):
    wid = _wid()
    base = wid * (CPT * C)
    last = base + (CPT - 1) * C

    pltpu.sync_copy(rowm.at[pl.ds(base, C)], ridx_a)
    pltpu.sync_copy(colm.at[pl.ds(base, C)], cidx_a)
    pltpu.async_copy(xr.at[ridx_a], buf_ra, sg1)
    pltpu.async_copy(xcb.at[cidx_a], buf_ca, sg2)

    def pair(k, _):
        e_a = base + (2 * k) * C
        e_b = e_a + C
        e_a2 = jnp.minimum(e_b + C, last)
        pltpu.sync_copy(rowm.at[pl.ds(e_b, C)], ridx_b)
        pltpu.sync_copy(colm.at[pl.ds(e_b, C)], cidx_b)
        pltpu.make_async_copy(xr.at[ridx_a], buf_ra, sg1).wait()
        pltpu.make_async_copy(xcb.at[cidx_a], buf_ca, sg2).wait()
        pltpu.async_copy(xr.at[ridx_b], buf_rb, sg3)
        pltpu.async_copy(xcb.at[cidx_b], buf_cb, sg4)
        cw1 = pltpu.async_copy(buf_ra, ga.at[pl.ds(e_a, C)], sw1)
        cw2 = pltpu.async_copy(buf_ca, gb.at[pl.ds(e_a, C)], sw2)
        cw1.wait()
        cw2.wait()
        pltpu.sync_copy(rowm.at[pl.ds(e_a2, C)], ridx_a)
        pltpu.sync_copy(colm.at[pl.ds(e_a2, C)], cidx_a)
        pltpu.async_copy(xr.at[ridx_a], buf_ra, sg1)
        pltpu.async_copy(xcb.at[cidx_a], buf_ca, sg2)
        pltpu.make_async_copy(xr.at[ridx_b], buf_rb, sg3).wait()
        pltpu.make_async_copy(xcb.at[cidx_b], buf_cb, sg4).wait()
        pltpu.sync_copy(buf_rb, ga.at[pl.ds(e_b, C)])
        pltpu.sync_copy(buf_cb, gb.at[pl.ds(e_b, C)])
        return 0
    lax.fori_loop(0, CPT // 2, pair, 0)

    pltpu.make_async_copy(xr.at[ridx_a], buf_ra, sg1).wait()
    pltpu.make_async_copy(xcb.at[cidx_a], buf_ca, sg2).wait()


_R = 2000


def _full(shape):
    nd = len(shape)
    return pl.BlockSpec(shape, lambda i, _n=nd: (0,) * _n)


def _t1_body(degp, inp, wa1, ba1, wa2, ba2, w2, hs_o, dis_o):
    deg = degp[0, :, 0:1] + degp[1, :, 0:1] + 1.0
    dis = lax.rsqrt(deg)
    ai = inp[:, 0:8]
    mf = inp[:, 8:]
    af = jnp.maximum(jnp.dot(ai, wa1[:], preferred_element_type=jnp.float32)
                     + ba1[:][None, :], 0.0)
    af = jnp.dot(af, wa2[:], preferred_element_type=jnp.float32) + ba2[:][None, :]
    x0 = jnp.concatenate([mf, af], axis=1)
    hs_o[:] = jnp.dot(x0, w2[:], preferred_element_type=jnp.float32) * dis
    dis_o[:] = dis


def _t1(deg_parts, inputs, Wa1, ba1, Wa2, ba2, W2):
    f = inputs.shape[1]
    return pl.pallas_call(
        _t1_body,
        grid=N // _R,
        in_specs=[
            pl.BlockSpec((2, _R, D), lambda i: (0, i, 0)),
            pl.BlockSpec((_R, f), lambda i: (i, 0)),
            _full(Wa1.shape), _full(ba1.shape), _full(Wa2.shape),
            _full(ba2.shape), _full(W2.shape),
        ],
        out_specs=[
            pl.BlockSpec((_R, D), lambda i: (i, 0)),
            pl.BlockSpec((_R, 1), lambda i: (i, 0)),
        ],
        out_shape=[
            jax.ShapeDtypeStruct((N, D), jnp.float32),
            jax.ShapeDtypeStruct((N, 1), jnp.float32),
        ],
    )(deg_parts, inputs, Wa1, ba1, Wa2, ba2, W2)


def _t2_body(segp, hs1, dis, b2, w3, hs2_o):
    seg = segp[0] + segp[1]
    x1 = jnp.maximum(dis[:] * (seg + hs1[:]) + b2[:][None, :], 0.0)
    hs2_o[:] = jnp.dot(x1, w3[:], preferred_element_type=jnp.float32) * dis[:]


def _t2(segp1, hs1, dis, b2, W3):
    return pl.pallas_call(
        _t2_body,
        grid=N // _R,
        in_specs=[
            pl.BlockSpec((2, _R, D), lambda i: (0, i, 0)),
            pl.BlockSpec((_R, D), lambda i: (i, 0)),
            pl.BlockSpec((_R, 1), lambda i: (i, 0)),
            _full(b2.shape), _full(W3.shape),
        ],
        out_specs=[pl.BlockSpec((_R, D), lambda i: (i, 0))],
        out_shape=[jax.ShapeDtypeStruct((N, D), jnp.float32)],
    )(segp1, hs1, dis, b2, W3)[0]


def _t3_body(segp, hs2, dis, inp, b3, wc1, bc1, wc2, bc2, wb1, bb1, a_m, b_m,
             we1, be1, x_o, pr_o, bx_o, xr_o, xc_o):
    seg = segp[0] + segp[1]
    x = dis[:] * (seg + hs2[:]) + b3[:][None, :]
    x_o[:] = x
    p1 = jnp.maximum(jnp.dot(x, wc1[:], preferred_element_type=jnp.float32)
                     + bc1[:][None, :], 0.0)
    pr_o[:] = jnp.dot(p1, wc2[:], preferred_element_type=jnp.float32) + bc2[:][None, :]
    q1 = jnp.maximum(jnp.dot(x, wb1[:], preferred_element_type=jnp.float32)
                     + bb1[:][None, :], 0.0)
    q2 = jnp.dot(jnp.dot(q1, a_m[:], preferred_element_type=jnp.float32),
                 b_m[:], preferred_element_type=jnp.float32)
    bx_o[:] = jnp.tanh(q2) + inp[:, 1:5]
    w_full = we1[:]
    xr_o[:] = jnp.dot(x, w_full[0:D, :], preferred_element_type=jnp.float32)
    xc_o[:] = (jnp.dot(x, w_full[D:, :], preferred_element_type=jnp.float32)
               + be1[:][None, :])


def _t3(segp2, hs2, dis, inputs, b3, Wc1, bc1, Wc2, bc2, Wb1, bb1, A, B,
        We1, be1):
    f = inputs.shape[1]
    return pl.pallas_call(
        _t3_body,
        grid=N // _R,
        in_specs=[
            pl.BlockSpec((2, _R, D), lambda i: (0, i, 0)),
            pl.BlockSpec((_R, D), lambda i: (i, 0)),
            pl.BlockSpec((_R, 1), lambda i: (i, 0)),
            pl.BlockSpec((_R, f), lambda i: (i, 0)),
            _full(b3.shape), _full(Wc1.shape), _full(bc1.shape),
            _full(Wc2.shape), _full(bc2.shape), _full(Wb1.shape),
            _full(bb1.shape), _full(A.shape), _full(B.shape),
            _full(We1.shape), _full(be1.shape),
        ],
        out_specs=[
            pl.BlockSpec((_R, D), lambda i: (i, 0)),
            pl.BlockSpec((_R, 10), lambda i: (i, 0)),
            pl.BlockSpec((_R, 4), lambda i: (i, 0)),
            pl.BlockSpec((_R, D), lambda i: (i, 0)),
            pl.BlockSpec((_R, D), lambda i: (i, 0)),
        ],
        out_shape=[
            jax.ShapeDtypeStruct((N, D), jnp.float32),
            jax.ShapeDtypeStruct((N, 10), jnp.float32),
            jax.ShapeDtypeStruct((N, 4), jnp.float32),
            jax.ShapeDtypeStruct((N, D), jnp.float32),
            jax.ShapeDtypeStruct((N, D), jnp.float32),
        ],
    )(segp2, hs2, dis, inputs, b3, Wc1, bc1, Wc2, bc2, Wb1, bb1, A, B,
      We1, be1)


_R4 = 4000


def _t4_body(ga, gb, we2, be2, we3, be3, out):
    h1 = jnp.maximum(ga[:] + gb[:], 0.0)
    h2 = jnp.maximum(jnp.dot(h1, we2[:], preferred_element_type=jnp.float32)
                     + be2[:][None, :], 0.0)
    z = jnp.dot(h2, we3[:], preferred_element_type=jnp.float32) + be3[:][None, :]
    out[:] = 1.0 / (1.0 + jnp.exp(-z))


def _t4(ga, gb, We2, be2, We3, be3):
    return pl.pallas_call(
        _t4_body,
        grid=E // _R4,
        in_specs=[
            pl.BlockSpec((_R4, D), lambda i: (i, 0)),
            pl.BlockSpec((_R4, D), lambda i: (i, 0)),
            _full(We2.shape), _full(be2.shape), _full(We3.shape),
            _full(be3.shape),
        ],
        out_specs=[pl.BlockSpec((_R4, 1), lambda i: (i, 0))],
        out_shape=[jax.ShapeDtypeStruct((E, 1), jnp.float32)],
    )(ga, gb, We2, be2, We3, be3)[0]


def kernel(inputs, edge_index, Wa1, ba1, Wa2, ba2, W2, b2, W3, b3, Wc1, bc1,
           Wc2, bc2, Wb1, bb1, A, B, We1, be1, We2, be2, We3, be3):
    src = edge_index[0]
    dst = edge_index[1]
    pad0 = jnp.zeros((EP - E,), jnp.int32)
    srcm = jnp.concatenate([src, pad0])
    dst_scat = jnp.concatenate([dst, jnp.full((EP - E,), N, jnp.int32)])
    dst_gath = jnp.concatenate([dst, pad0])
    zin = jnp.zeros((NP, D), jnp.float32)
    ones_in = jnp.ones((2 * C, D), jnp.float32)

    deg_parts = _deg_sc(dst_scat, zin, ones_in)
    hs1, dis = _t1(deg_parts, inputs, Wa1, ba1, Wa2, ba2, W2)
    segp1 = _seg_sc(hs1, srcm, dst_scat, zin)
    hs2 = _t2(segp1, hs1, dis, b2, W3)
    segp2 = _seg_sc(hs2, srcm, dst_scat, zin)
    x, predict, box, xr, xc = _t3(segp2, hs2, dis, inputs, b3, Wc1, bc1,
                                  Wc2, bc2, Wb1, bb1, A, B, We1, be1)
    ga, gb = _edge_sc(xr, xc, srcm, dst_gath)
    edge = _t4(ga, gb, We2, be2, We3, be3)
    return predict, box, edge, x
